# baseline (speedup 1.0000x reference)
import functools

import jax
import jax.numpy as jnp
from jax import lax
from jax.experimental import pallas as pl
from jax.experimental.pallas import tpu as pltpu
from jax.experimental.pallas import tpu_sc as plsc

NC = 2
NS = 16
LANES = 16
NW = NC * NS
CB = 64
NRB = 4
NIB = 8
N_PAD = 10240
ROWS_PER_SUB = N_PAD // NS
ZCHUNKS = ROWS_PER_SUB // CB


_SKIP_GATHER = False
_SKIP_SCATTER = True


def _sc_mesh():
  return plsc.VectorSubcoreMesh(
      core_axis_name="c", subcore_axis_name="s", num_cores=NC, num_subcores=NS)


def _make_spmv(n_rows, k, chunks_per_worker, exp_coef, emit_den,
               interpret=False):
  nch = chunks_per_worker
  assert nch % NIB == 0
  out_type = [jax.ShapeDtypeStruct((NC, N_PAD, k), jnp.float32)]
  if emit_den:
    out_type += [jax.ShapeDtypeStruct((NW, N_PAD), jnp.float32)]
  scratch = (
      [pltpu.VMEM((CB, k), jnp.float32)] * NRB
      + [pltpu.VMEM((3, CB), jnp.int32)] * NIB
      + [pltpu.VMEM_SHARED((N_PAD, k), jnp.float32)]
      + [pltpu.SemaphoreType.DMA] * (2 * NRB + NIB)
  )
  if emit_den:
    scratch += [pltpu.VMEM((N_PAD,), jnp.float32)]

  def body(g_hbm, eidx_hbm, *rest):
    if emit_den:
      part_hbm, den_hbm = rest[0], rest[1]
      rest = rest[2:]
      dpriv = rest[-1]
    else:
      part_hbm = rest[0]
      rest = rest[1:]
    rows = rest[0:NRB]
    ib = rest[NRB:NRB + NIB]
    acc = rest[NRB + NIB]
    semg = rest[NRB + NIB + 1:NRB + NIB + 1 + NRB]
    sems = rest[NRB + NIB + 1 + NRB:NRB + NIB + 1 + 2 * NRB]
    semi = rest[NRB + NIB + 1 + 2 * NRB:NRB + NIB + 1 + 2 * NRB + NIB]
    rows0 = rows[0]
    cid = lax.axis_index("c")
    sid = lax.axis_index("s")
    wid = sid * NC + cid
    base = wid * nch

    def zrow(i, _):
      for j in range(k // LANES):
        rows0[i, pl.ds(j * LANES, LANES)] = jnp.zeros((LANES,), jnp.float32)
      return 0
    lax.fori_loop(0, CB, zrow, 0)
    for z in range(ZCHUNKS):
      r0 = sid * ROWS_PER_SUB + z * CB
      pltpu.sync_copy(rows0, acc.at[pl.ds(r0, CB)])
    if emit_den:
      def zd(i, _):
        dpriv[pl.ds(i * LANES, LANES)] = jnp.zeros((LANES,), jnp.float32)
        return 0
      lax.fori_loop(0, N_PAD // LANES, zd, 0)
    plsc.subcore_barrier()

    def start_idx(ci, q):
      pltpu.async_copy(eidx_hbm.at[base + ci], ib[q], semi[q])
    def wait_idx(q):
      pltpu.make_async_copy(eidx_hbm.at[base], ib[q], semi[q]).wait()
    def start_gather(b, q):
      if not _SKIP_GATHER:
        pltpu.async_copy(g_hbm.at[ib[q].at[0]], rows[b], semg[b])
    def wait_gather(b, q):
      if not _SKIP_GATHER:
        pltpu.make_async_copy(g_hbm.at[ib[q].at[0]], rows[b], semg[b]).wait()
    def start_scatter(b, q):
      if not _SKIP_SCATTER:
        pltpu.async_copy(rows[b], acc.at[ib[q].at[1]], sems[b], add=True)
    def wait_scatter(b, q):
      if not _SKIP_SCATTER:
        pltpu.make_async_copy(rows[b], acc.at[ib[q].at[1]], sems[b]).wait()

    def scale(b, q):
      buf = rows[b]
      idxq = ib[q]
      def sgroup(gi, _):
        sl = pl.ds(gi * LANES, LANES)
        cvec = plsc.bitcast(idxq[2, sl], jnp.float32)
        if exp_coef:
          cvec = jnp.exp(cvec)
        if emit_den:
          plsc.addupdate_scatter(dpriv, [idxq[1, sl]], cvec)
        for i in range(LANES):
          cc = cvec[i]
          for j in range(k // LANES):
            fsl = pl.ds(j * LANES, LANES)
            buf[gi * LANES + i, fsl] = buf[gi * LANES + i, fsl] * cc
        return 0
      lax.fori_loop(0, CB // LANES, sgroup, 0)

    for q0 in range(5):
      start_idx(q0, q0)
    wait_idx(0)
    start_gather(0, 0)

    ngrp = nch // NIB
    def group(g8, _):
      for r in range(NIB):
        ci = NIB * g8 + r
        b, q = r % NRB, r
        wait_gather(b, q)
        if r == NIB - 1:
          @pl.when(g8 < ngrp - 1)
          def _():
            wait_idx((r + 1) % NIB)
        else:
          wait_idx((r + 1) % NIB)
        if r < 3:
          @pl.when(g8 > 0)
          def _():
            wait_scatter((r - 3) % NRB, (r - 3) % NIB)
        else:
          wait_scatter((r - 3) % NRB, (r - 3) % NIB)
        if r == NIB - 1:
          @pl.when(g8 < ngrp - 1)
          def _():
            start_gather((r + 1) % NRB, (r + 1) % NIB)
        else:
          start_gather((r + 1) % NRB, (r + 1) % NIB)
        if r < 3:
          start_idx(ci + 5, (r + 5) % NIB)
        else:
          @pl.when(g8 < ngrp - 1)
          def _():
            start_idx(ci + 5, (r + 5) % NIB)
        scale(b, q)
        start_scatter(b, q)
      return 0
    lax.fori_loop(0, ngrp, group, 0)
    for t in range(3):
      ci = nch - 3 + t
      wait_scatter(ci % NRB, ci % NIB)
    plsc.subcore_barrier()

    if emit_den:
      pltpu.sync_copy(dpriv, den_hbm.at[wid])
    for z in range(ZCHUNKS):
      r0 = sid * ROWS_PER_SUB + z * CB
      pltpu.sync_copy(acc.at[pl.ds(r0, CB)], rows0)
      pltpu.sync_copy(rows0, part_hbm.at[cid, pl.ds(r0, CB)])

  return pl.kernel(body, out_type, mesh=_sc_mesh(), scratch_types=scratch,
                   compiler_params=pltpu.CompilerParams(
                       use_tc_tiling_on_sc=False, needs_layout_passes=False),
                   interpret=interpret)


def _tc_matmul(x, w, interpret=False):
  m, d = x.shape
  h = w.shape[1]
  bm = 400
  def body(x_ref, w_ref, o_ref):
    o_ref[...] = jnp.dot(x_ref[...], w_ref[...],
                         preferred_element_type=jnp.float32)
  return pl.pallas_call(
      body,
      grid=(m // bm,),
      in_specs=[pl.BlockSpec((bm, d), lambda i: (i, 0)),
                pl.BlockSpec((d, h), lambda i: (0, 0))],
      out_specs=pl.BlockSpec((bm, h), lambda i: (i, 0)),
      out_shape=jax.ShapeDtypeStruct((m, h), jnp.float32),
      interpret=interpret)(x, w)


def _tc_norm_relu_matmul(p, dinv, b, w, n, interpret=False):
  k = p.shape[2]
  h = w.shape[1]
  bm = 400
  def body(p_ref, d_ref, b_ref, w_ref, o_ref):
    ps = p_ref[0] + p_ref[1]
    hh = jnp.maximum(ps * d_ref[...] + b_ref[...], 0.0)
    o_ref[...] = jnp.dot(hh, w_ref[...], preferred_element_type=jnp.float32)
  return pl.pallas_call(
      body,
      grid=(n // bm,),
      in_specs=[pl.BlockSpec((NC, bm, k), lambda i: (0, i, 0)),
                pl.BlockSpec((bm, 1), lambda i: (i, 0)),
                pl.BlockSpec((1, k), lambda i: (0, 0)),
                pl.BlockSpec((k, h), lambda i: (0, 0))],
      out_specs=pl.BlockSpec((bm, h), lambda i: (i, 0)),
      out_shape=jax.ShapeDtypeStruct((n, h), jnp.float32),
      interpret=interpret)(p, dinv, b, w)


def _tc_norm_bias(p, dinv, b, n, interpret=False):
  k = p.shape[2]
  bm = 400
  def body(p_ref, d_ref, b_ref, o_ref):
    o_ref[...] = (p_ref[0] + p_ref[1]) * d_ref[...] + b_ref[...]
  return pl.pallas_call(
      body,
      grid=(n // bm,),
      in_specs=[pl.BlockSpec((NC, bm, k), lambda i: (0, i, 0)),
                pl.BlockSpec((bm, 1), lambda i: (i, 0)),
                pl.BlockSpec((1, k), lambda i: (0, 0))],
      out_specs=pl.BlockSpec((bm, k), lambda i: (i, 0)),
      out_shape=jax.ShapeDtypeStruct((n, k), jnp.float32),
      interpret=interpret)(p, dinv, b)


def _tc_recip(den, n, interpret=False):
  bm = 400
  nw = den.shape[0]
  def body(d_ref, o_ref):
    d = jnp.sum(d_ref[...], axis=0)
    o_ref[...] = jnp.where(d > 0, 1.0 / d, 0.0)
  return pl.pallas_call(
      body,
      grid=(n // bm,),
      in_specs=[pl.BlockSpec((nw, bm, 1), lambda i: (0, i, 0))],
      out_specs=pl.BlockSpec((bm, 1), lambda i: (i, 0)),
      out_shape=jax.ShapeDtypeStruct((n, 1), jnp.float32),
      interpret=interpret)(den)


def _tc_lpa_blend(r, h3, n, interpret=False):
  k = h3.shape[1]
  bm = 400
  def body(r_ref, h_ref, o_ref):
    o_ref[...] = 0.9 * (r_ref[0] + r_ref[1]) + 0.1 * h_ref[...]
  return pl.pallas_call(
      body,
      grid=(n // bm,),
      in_specs=[pl.BlockSpec((NC, bm, k), lambda i: (0, i, 0)),
                pl.BlockSpec((bm, k), lambda i: (i, 0))],
      out_specs=pl.BlockSpec((bm, k), lambda i: (i, 0)),
      out_shape=jax.ShapeDtypeStruct((n, k), jnp.float32),
      interpret=interpret)(r, h3)


def _forward(features, edge_index, lpa_adj, W1, b1, W2, b2, W3, b3,
             interpret=False):
  n, d = features.shape
  e = edge_index.shape[1]
  h = W1.shape[1]
  c = W3.shape[1]

  grain = NW * CB * 8
  e_pad = ((e + grain - 1) // grain) * grain
  pad = e_pad - e
  src = edge_index[0]
  dst = edge_index[1]
  lv = lpa_adj[:, 0]
  if pad:
    src = jnp.concatenate([src, jnp.zeros((pad,), jnp.int32)])
    dst = jnp.concatenate([dst, jnp.full((pad,), n, jnp.int32)])
    lv = jnp.concatenate([lv, jnp.zeros((pad,), jnp.float32)])
  nch = e_pad // (NW * CB)
  eidx = jnp.stack(
      [src.reshape(nch * NW, CB), dst.reshape(nch * NW, CB),
       lax.bitcast_convert_type(lv, jnp.int32).reshape(nch * NW, CB)],
      axis=1)

  spmv_ex = _make_spmv(n, h, nch, True, True, interpret)
  spmv_h = _make_spmv(n, h, nch, True, False, interpret)
  spmv_c = _make_spmv(n, c, nch, True, False, interpret)
  spmv_raw = _make_spmv(n, c, nch, False, False, interpret)

  t0 = _tc_matmul(features, W1, interpret)
  p1, den = spmv_ex(t0, eidx)
  dinv = _tc_recip(den.reshape(NW, N_PAD, 1)[:, :n], n, interpret)
  t1 = _tc_norm_relu_matmul(p1, dinv, b1.reshape(1, h), W2, n, interpret)
  p2 = spmv_h(t1, eidx)[0]
  t2 = _tc_norm_relu_matmul(p2, dinv, b2.reshape(1, h), W3, n, interpret)
  p3 = spmv_c(t2, eidx)[0]
  h3 = _tc_norm_bias(p3, dinv, b3.reshape(1, c), n, interpret)
  r = spmv_raw(h3, eidx)[0]
  z = _tc_lpa_blend(r, h3, n, interpret)
  return h3, z


def kernel(features, edge_index, lpa_adj, W1, b1, W2, b2, W3, b3):
  return _forward(features, edge_index, lpa_adj, W1, b1, W2, b2, W3, b3)


# speedup vs baseline: 1.2127x; 1.2127x over previous
import jax
import jax.numpy as jnp
from jax import lax
from jax.experimental import pallas as pl
from jax.experimental.pallas import tpu as pltpu
from jax.experimental.pallas import tpu_sc as plsc

NC = 2
NS = 16
LANES = 16
NW = NC * NS
CB = 64
NRB = 4
NIB = 8
HALF = 64
N_PAD = 10240
ROWS_PER_SUB = N_PAD // NS
ZCHUNKS = ROWS_PER_SUB // CB


def _sc_mesh():
  return plsc.VectorSubcoreMesh(
      core_axis_name="c", subcore_axis_name="s", num_cores=NC, num_subcores=NS)


def _make_spmv(n_rows, n_halves, chunks_per_worker, exp_coef, emit_den,
               interpret=False):
  nch = chunks_per_worker
  assert nch % NIB == 0
  out_type = [jax.ShapeDtypeStruct((NC, n_halves, N_PAD, HALF), jnp.float32)]
  if emit_den:
    out_type += [jax.ShapeDtypeStruct((NW, N_PAD), jnp.float32)]
  scratch = (
      [pltpu.VMEM((CB, HALF), jnp.float32)] * NRB
      + [pltpu.VMEM((3, CB), jnp.int32)] * NIB
      + [pltpu.VMEM_SHARED((N_PAD, HALF), jnp.float32)]
      + [pltpu.VMEM_SHARED((N_PAD, HALF), jnp.float32)]
      + [pltpu.SemaphoreType.DMA] * (2 * NRB + NIB)
  )
  if emit_den:
    scratch += [pltpu.VMEM((N_PAD,), jnp.float32)]

  def body(*args):
    g_halves = args[0:n_halves]
    eidx_hbm = args[n_halves]
    part_hbm = args[n_halves + 1]
    if emit_den:
      den_hbm = args[n_halves + 2]
      rest = args[n_halves + 3:]
      dpriv = rest[-1]
    else:
      rest = args[n_halves + 2:]
    rows = rest[0:NRB]
    ib = rest[NRB:NRB + NIB]
    gtab = rest[NRB + NIB]
    acc = rest[NRB + NIB + 1]
    s0 = NRB + NIB + 2
    semg = rest[s0:s0 + NRB]
    sems = rest[s0 + NRB:s0 + 2 * NRB]
    semi = rest[s0 + 2 * NRB:s0 + 2 * NRB + NIB]
    rows0 = rows[0]
    cid = lax.axis_index("c")
    sid = lax.axis_index("s")
    wid = sid * NC + cid
    base = wid * nch

    def start_idx(ci, q):
      pltpu.async_copy(eidx_hbm.at[base + ci], ib[q], semi[q])
    def wait_idx(q):
      pltpu.make_async_copy(eidx_hbm.at[base], ib[q], semi[q]).wait()
    def start_gather(b, q):
      pltpu.async_copy(gtab.at[ib[q].at[0]], rows[b], semg[b])
    def wait_gather(b, q):
      pltpu.make_async_copy(gtab.at[ib[q].at[0]], rows[b], semg[b]).wait()
    def start_scatter(b, q):
      pltpu.async_copy(rows[b], acc.at[ib[q].at[1]], sems[b], add=True)
    def wait_scatter(b, q):
      pltpu.make_async_copy(rows[b], acc.at[ib[q].at[1]], sems[b]).wait()

    def scale(b, q, den_flag):
      buf = rows[b]
      idxq = ib[q]
      def sgroup(gi, _):
        sl = pl.ds(gi * LANES, LANES)
        cvec = plsc.bitcast(idxq[2, sl], jnp.float32)
        if exp_coef:
          cvec = jnp.exp(cvec)
        if den_flag:
          plsc.addupdate_scatter(dpriv, [idxq[1, sl]], cvec)
        for i in range(LANES):
          cc = cvec[i]
          for j in range(HALF // LANES):
            fsl = pl.ds(j * LANES, LANES)
            buf[gi * LANES + i, fsl] = buf[gi * LANES + i, fsl] * cc
        return 0
      lax.fori_loop(0, CB // LANES, sgroup, 0)

    def half_pass(half, den_flag):
      g_hbm = g_halves[half]
      def zrow(i, _):
        for j in range(HALF // LANES):
          rows0[i, pl.ds(j * LANES, LANES)] = jnp.zeros((LANES,), jnp.float32)
        return 0
      lax.fori_loop(0, CB, zrow, 0)
      for z in range(ZCHUNKS):
        r0 = sid * ROWS_PER_SUB + z * CB
        pltpu.sync_copy(rows0, acc.at[pl.ds(r0, CB)])
      @pl.when(sid < NS - 1)
      def _():
        r0 = sid * ROWS_PER_SUB
        pltpu.sync_copy(g_hbm.at[pl.ds(r0, ROWS_PER_SUB)],
                        gtab.at[pl.ds(r0, ROWS_PER_SUB)])
      @pl.when(sid == NS - 1)
      def _():
        tail = n_rows - (NS - 1) * ROWS_PER_SUB
        pltpu.sync_copy(g_hbm.at[pl.ds((NS - 1) * ROWS_PER_SUB, tail)],
                        gtab.at[pl.ds((NS - 1) * ROWS_PER_SUB, tail)])
      if den_flag:
        def zd(i, _):
          dpriv[pl.ds(i * LANES, LANES)] = jnp.zeros((LANES,), jnp.float32)
          return 0
        lax.fori_loop(0, N_PAD // LANES, zd, 0)
      plsc.subcore_barrier()

      for q0 in range(5):
        start_idx(q0, q0)
      wait_idx(0)
      start_gather(0, 0)
      ngrp = nch // NIB
      def group(g8, _):
        for r in range(NIB):
          ci = NIB * g8 + r
          b, q = r % NRB, r
          wait_gather(b, q)
          if r == NIB - 1:
            @pl.when(g8 < ngrp - 1)
            def _():
              wait_idx((r + 1) % NIB)
          else:
            wait_idx((r + 1) % NIB)
          if r < 3:
            @pl.when(g8 > 0)
            def _():
              wait_scatter((r - 3) % NRB, (r - 3) % NIB)
          else:
            wait_scatter((r - 3) % NRB, (r - 3) % NIB)
          if r == NIB - 1:
            @pl.when(g8 < ngrp - 1)
            def _():
              start_gather((r + 1) % NRB, (r + 1) % NIB)
          else:
            start_gather((r + 1) % NRB, (r + 1) % NIB)
          if r < 3:
            start_idx(ci + 5, (r + 5) % NIB)
          else:
            @pl.when(g8 < ngrp - 1)
            def _():
              start_idx(ci + 5, (r + 5) % NIB)
          scale(b, q, den_flag)
          start_scatter(b, q)
        return 0
      lax.fori_loop(0, ngrp, group, 0)
      for t in range(3):
        ci = nch - 3 + t
        wait_scatter(ci % NRB, ci % NIB)
      plsc.subcore_barrier()

      for z in range(ZCHUNKS):
        r0 = sid * ROWS_PER_SUB + z * CB
        pltpu.sync_copy(acc.at[pl.ds(r0, CB)], rows0)
        pltpu.sync_copy(rows0, part_hbm.at[cid, half, pl.ds(r0, CB)])

    for half in range(n_halves):
      half_pass(half, emit_den and half == 0)
    if emit_den:
      pltpu.sync_copy(dpriv, den_hbm.at[wid])

  return pl.kernel(body, out_type, mesh=_sc_mesh(), scratch_types=scratch,
                   compiler_params=pltpu.CompilerParams(
                       use_tc_tiling_on_sc=False, needs_layout_passes=False),
                   interpret=interpret)


def _tc_matmul_split(x, w, interpret=False):
  m, d = x.shape
  h = w.shape[1]
  bm = 400
  def body(x_ref, w_ref, oa_ref, ob_ref):
    o = jnp.dot(x_ref[...], w_ref[...], preferred_element_type=jnp.float32)
    oa_ref[...] = o[:, 0:HALF]
    ob_ref[...] = o[:, HALF:2 * HALF]
  return pl.pallas_call(
      body,
      grid=(m // bm,),
      in_specs=[pl.BlockSpec((bm, d), lambda i: (i, 0)),
                pl.BlockSpec((d, h), lambda i: (0, 0))],
      out_specs=[pl.BlockSpec((bm, HALF), lambda i: (i, 0)),
                 pl.BlockSpec((bm, HALF), lambda i: (i, 0))],
      out_shape=[jax.ShapeDtypeStruct((m, HALF), jnp.float32),
                 jax.ShapeDtypeStruct((m, HALF), jnp.float32)],
      interpret=interpret)(x, w)


def _tc_norm_relu_matmul(p, dinv, b, w, n, split_out, interpret=False):
  nh = p.shape[1]
  h = w.shape[1]
  bm = 400
  def body(p_ref, d_ref, b_ref, w_ref, *outs):
    o = None
    for hf in range(nh):
      ps = p_ref[0, hf] + p_ref[1, hf]
      hh = jnp.maximum(ps * d_ref[...] + b_ref[:, hf * HALF:(hf + 1) * HALF],
                       0.0)
      t = jnp.dot(hh, w_ref[hf * HALF:(hf + 1) * HALF, :],
                  preferred_element_type=jnp.float32)
      o = t if o is None else o + t
    if split_out:
      outs[0][...] = o[:, 0:HALF]
      outs[1][...] = o[:, HALF:2 * HALF]
    else:
      outs[0][...] = o
  if split_out:
    out_specs = [pl.BlockSpec((bm, HALF), lambda i: (i, 0)),
                 pl.BlockSpec((bm, HALF), lambda i: (i, 0))]
    out_shape = [jax.ShapeDtypeStruct((n, HALF), jnp.float32),
                 jax.ShapeDtypeStruct((n, HALF), jnp.float32)]
  else:
    out_specs = [pl.BlockSpec((bm, h), lambda i: (i, 0))]
    out_shape = [jax.ShapeDtypeStruct((n, h), jnp.float32)]
  return pl.pallas_call(
      body,
      grid=(n // bm,),
      in_specs=[pl.BlockSpec((NC, nh, bm, HALF), lambda i: (0, 0, i, 0)),
                pl.BlockSpec((bm, 1), lambda i: (i, 0)),
                pl.BlockSpec((1, nh * HALF), lambda i: (0, 0)),
                pl.BlockSpec((nh * HALF, h), lambda i: (0, 0))],
      out_specs=out_specs,
      out_shape=out_shape,
      interpret=interpret)(p, dinv, b, w)


def _tc_norm_bias(p, dinv, b, n, interpret=False):
  k = p.shape[3]
  bm = 400
  def body(p_ref, d_ref, b_ref, o_ref):
    o_ref[...] = (p_ref[0, 0] + p_ref[1, 0]) * d_ref[...] + b_ref[...]
  return pl.pallas_call(
      body,
      grid=(n // bm,),
      in_specs=[pl.BlockSpec((NC, 1, bm, k), lambda i: (0, 0, i, 0)),
                pl.BlockSpec((bm, 1), lambda i: (i, 0)),
                pl.BlockSpec((1, k), lambda i: (0, 0))],
      out_specs=pl.BlockSpec((bm, k), lambda i: (i, 0)),
      out_shape=jax.ShapeDtypeStruct((n, k), jnp.float32),
      interpret=interpret)(p, dinv, b)


def _tc_recip(den, n, interpret=False):
  bm = 400
  nw = den.shape[0]
  def body(d_ref, o_ref):
    d = jnp.sum(d_ref[...], axis=0)
    o_ref[...] = jnp.where(d > 0, 1.0 / d, 0.0)
  return pl.pallas_call(
      body,
      grid=(n // bm,),
      in_specs=[pl.BlockSpec((nw, bm, 1), lambda i: (0, i, 0))],
      out_specs=pl.BlockSpec((bm, 1), lambda i: (i, 0)),
      out_shape=jax.ShapeDtypeStruct((n, 1), jnp.float32),
      interpret=interpret)(den)


def _tc_lpa_blend(r, h3, n, interpret=False):
  k = h3.shape[1]
  bm = 400
  def body(r_ref, h_ref, o_ref):
    o_ref[...] = 0.9 * (r_ref[0, 0] + r_ref[1, 0]) + 0.1 * h_ref[...]
  return pl.pallas_call(
      body,
      grid=(n // bm,),
      in_specs=[pl.BlockSpec((NC, 1, bm, k), lambda i: (0, 0, i, 0)),
                pl.BlockSpec((bm, k), lambda i: (i, 0))],
      out_specs=pl.BlockSpec((bm, k), lambda i: (i, 0)),
      out_shape=jax.ShapeDtypeStruct((n, k), jnp.float32),
      interpret=interpret)(r, h3)


def _forward(features, edge_index, lpa_adj, W1, b1, W2, b2, W3, b3,
             interpret=False):
  n, d = features.shape
  e = edge_index.shape[1]
  h = W1.shape[1]
  c = W3.shape[1]

  grain = NW * CB * NIB
  e_pad = ((e + grain - 1) // grain) * grain
  pad = e_pad - e
  src = edge_index[0]
  dst = edge_index[1]
  lv = lpa_adj[:, 0]
  if pad:
    src = jnp.concatenate([src, jnp.zeros((pad,), jnp.int32)])
    dst = jnp.concatenate([dst, jnp.full((pad,), n, jnp.int32)])
    lv = jnp.concatenate([lv, jnp.zeros((pad,), jnp.float32)])
  nch = e_pad // (NW * CB)
  eidx = jnp.stack(
      [src.reshape(nch * NW, CB), dst.reshape(nch * NW, CB),
       lax.bitcast_convert_type(lv, jnp.int32).reshape(nch * NW, CB)],
      axis=1)

  spmv_ex = _make_spmv(n, 2, nch, True, True, interpret)
  spmv_h = _make_spmv(n, 2, nch, True, False, interpret)
  spmv_c = _make_spmv(n, 1, nch, True, False, interpret)
  spmv_raw = _make_spmv(n, 1, nch, False, False, interpret)

  t0a, t0b = _tc_matmul_split(features, W1, interpret)
  p1, den = spmv_ex(t0a, t0b, eidx)
  dinv = _tc_recip(den.reshape(NW, N_PAD, 1)[:, :n], n, interpret)
  t1a, t1b = _tc_norm_relu_matmul(p1, dinv, b1.reshape(1, h), W2, n, True,
                                  interpret)
  p2 = spmv_h(t1a, t1b, eidx)[0]
  t2 = _tc_norm_relu_matmul(p2, dinv, b2.reshape(1, h), W3, n, False,
                            interpret)[0]
  p3 = spmv_c(t2, eidx)[0]
  h3 = _tc_norm_bias(p3, dinv, b3.reshape(1, c), n, interpret)
  r = spmv_raw(h3, eidx)[0]
  z = _tc_lpa_blend(r, h3, n, interpret)
  return h3, z


def kernel(features, edge_index, lpa_adj, W1, b1, W2, b2, W3, b3):
  return _forward(features, edge_index, lpa_adj, W1, b1, W2, b2, W3, b3)


# speedup vs baseline: 1.2391x; 1.0218x over previous
import jax
import jax.numpy as jnp
from jax import lax
from jax.experimental import pallas as pl
from jax.experimental.pallas import tpu as pltpu
from jax.experimental.pallas import tpu_sc as plsc

NC = 2
NS = 16
LANES = 16
NW = NC * NS
CB = 128
NRB = 4
NIB = 8
HALF = 64
N_PAD = 10240
ROWS_PER_SUB = N_PAD // NS
ZCHUNKS = ROWS_PER_SUB // CB


def _sc_mesh():
  return plsc.VectorSubcoreMesh(
      core_axis_name="c", subcore_axis_name="s", num_cores=NC, num_subcores=NS)


def _make_spmv(n_rows, n_halves, chunks_per_worker, exp_coef, emit_den,
               interpret=False):
  nch = chunks_per_worker
  assert nch % NIB == 0
  out_type = [jax.ShapeDtypeStruct((NC, n_halves, N_PAD, HALF), jnp.float32)]
  if emit_den:
    out_type += [jax.ShapeDtypeStruct((NW, N_PAD), jnp.float32)]
  scratch = (
      [pltpu.VMEM((CB, HALF), jnp.float32)] * NRB
      + [pltpu.VMEM((3, CB), jnp.int32)] * NIB
      + [pltpu.VMEM_SHARED((N_PAD, HALF), jnp.float32)]
      + [pltpu.VMEM_SHARED((N_PAD, HALF), jnp.float32)]
      + [pltpu.SemaphoreType.DMA] * (2 * NRB + NIB)
  )
  if emit_den:
    scratch += [pltpu.VMEM((N_PAD,), jnp.float32)]

  def body(*args):
    g_halves = args[0:n_halves]
    eidx_hbm = args[n_halves]
    part_hbm = args[n_halves + 1]
    if emit_den:
      den_hbm = args[n_halves + 2]
      rest = args[n_halves + 3:]
      dpriv = rest[-1]
    else:
      rest = args[n_halves + 2:]
    rows = rest[0:NRB]
    ib = rest[NRB:NRB + NIB]
    gtab = rest[NRB + NIB]
    acc = rest[NRB + NIB + 1]
    s0 = NRB + NIB + 2
    semg = rest[s0:s0 + NRB]
    sems = rest[s0 + NRB:s0 + 2 * NRB]
    semi = rest[s0 + 2 * NRB:s0 + 2 * NRB + NIB]
    rows0 = rows[0]
    cid = lax.axis_index("c")
    sid = lax.axis_index("s")
    wid = sid * NC + cid
    base = wid * nch

    def start_idx(ci, q):
      pltpu.async_copy(eidx_hbm.at[base + ci], ib[q], semi[q])
    def wait_idx(q):
      pltpu.make_async_copy(eidx_hbm.at[base], ib[q], semi[q]).wait()
    def start_gather(b, q):
      pltpu.async_copy(gtab.at[ib[q].at[0]], rows[b], semg[b])
    def wait_gather(b, q):
      pltpu.make_async_copy(gtab.at[ib[q].at[0]], rows[b], semg[b]).wait()
    def start_scatter(b, q):
      pltpu.async_copy(rows[b], acc.at[ib[q].at[1]], sems[b], add=True)
    def wait_scatter(b, q):
      pltpu.make_async_copy(rows[b], acc.at[ib[q].at[1]], sems[b]).wait()

    def scale(b, q, den_flag):
      buf = rows[b]
      idxq = ib[q]
      def sgroup(gi, _):
        sl = pl.ds(gi * LANES, LANES)
        cvec = plsc.bitcast(idxq[2, sl], jnp.float32)
        if exp_coef:
          cvec = jnp.exp(cvec)
        if den_flag:
          plsc.addupdate_scatter(dpriv, [idxq[1, sl]], cvec)
        for i in range(LANES):
          cc = cvec[i]
          for j in range(HALF // LANES):
            fsl = pl.ds(j * LANES, LANES)
            buf[gi * LANES + i, fsl] = buf[gi * LANES + i, fsl] * cc
        return 0
      lax.fori_loop(0, CB // LANES, sgroup, 0)

    def half_pass(half, den_flag):
      g_hbm = g_halves[half]
      def zrow(i, _):
        for j in range(HALF // LANES):
          rows0[i, pl.ds(j * LANES, LANES)] = jnp.zeros((LANES,), jnp.float32)
        return 0
      lax.fori_loop(0, CB, zrow, 0)
      for z in range(ZCHUNKS):
        r0 = sid * ROWS_PER_SUB + z * CB
        pltpu.sync_copy(rows0, acc.at[pl.ds(r0, CB)])
      @pl.when(sid < NS - 1)
      def _():
        r0 = sid * ROWS_PER_SUB
        pltpu.sync_copy(g_hbm.at[pl.ds(r0, ROWS_PER_SUB)],
                        gtab.at[pl.ds(r0, ROWS_PER_SUB)])
      @pl.when(sid == NS - 1)
      def _():
        tail = n_rows - (NS - 1) * ROWS_PER_SUB
        pltpu.sync_copy(g_hbm.at[pl.ds((NS - 1) * ROWS_PER_SUB, tail)],
                        gtab.at[pl.ds((NS - 1) * ROWS_PER_SUB, tail)])
      if den_flag:
        def zd(i, _):
          dpriv[pl.ds(i * LANES, LANES)] = jnp.zeros((LANES,), jnp.float32)
          return 0
        lax.fori_loop(0, N_PAD // LANES, zd, 0)
      plsc.subcore_barrier()

      for q0 in range(5):
        start_idx(q0, q0)
      wait_idx(0)
      start_gather(0, 0)
      ngrp = nch // NIB
      def group(g8, _):
        for r in range(NIB):
          ci = NIB * g8 + r
          b, q = r % NRB, r
          wait_gather(b, q)
          if r == NIB - 1:
            @pl.when(g8 < ngrp - 1)
            def _():
              wait_idx((r + 1) % NIB)
          else:
            wait_idx((r + 1) % NIB)
          if r < 3:
            @pl.when(g8 > 0)
            def _():
              wait_scatter((r - 3) % NRB, (r - 3) % NIB)
          else:
            wait_scatter((r - 3) % NRB, (r - 3) % NIB)
          if r == NIB - 1:
            @pl.when(g8 < ngrp - 1)
            def _():
              start_gather((r + 1) % NRB, (r + 1) % NIB)
          else:
            start_gather((r + 1) % NRB, (r + 1) % NIB)
          if r < 3:
            start_idx(ci + 5, (r + 5) % NIB)
          else:
            @pl.when(g8 < ngrp - 1)
            def _():
              start_idx(ci + 5, (r + 5) % NIB)
          scale(b, q, den_flag)
          start_scatter(b, q)
        return 0
      lax.fori_loop(0, ngrp, group, 0)
      for t in range(3):
        ci = nch - 3 + t
        wait_scatter(ci % NRB, ci % NIB)
      plsc.subcore_barrier()

      for z in range(ZCHUNKS):
        r0 = sid * ROWS_PER_SUB + z * CB
        pltpu.sync_copy(acc.at[pl.ds(r0, CB)], rows0)
        pltpu.sync_copy(rows0, part_hbm.at[cid, half, pl.ds(r0, CB)])

    for half in range(n_halves):
      half_pass(half, emit_den and half == 0)
    if emit_den:
      pltpu.sync_copy(dpriv, den_hbm.at[wid])

  return pl.kernel(body, out_type, mesh=_sc_mesh(), scratch_types=scratch,
                   compiler_params=pltpu.CompilerParams(
                       use_tc_tiling_on_sc=False, needs_layout_passes=False),
                   interpret=interpret)


def _tc_matmul_split(x, w, interpret=False):
  m, d = x.shape
  h = w.shape[1]
  bm = 400
  def body(x_ref, w_ref, oa_ref, ob_ref):
    o = jnp.dot(x_ref[...], w_ref[...], preferred_element_type=jnp.float32)
    oa_ref[...] = o[:, 0:HALF]
    ob_ref[...] = o[:, HALF:2 * HALF]
  return pl.pallas_call(
      body,
      grid=(m // bm,),
      in_specs=[pl.BlockSpec((bm, d), lambda i: (i, 0)),
                pl.BlockSpec((d, h), lambda i: (0, 0))],
      out_specs=[pl.BlockSpec((bm, HALF), lambda i: (i, 0)),
                 pl.BlockSpec((bm, HALF), lambda i: (i, 0))],
      out_shape=[jax.ShapeDtypeStruct((m, HALF), jnp.float32),
                 jax.ShapeDtypeStruct((m, HALF), jnp.float32)],
      interpret=interpret)(x, w)


def _tc_norm_relu_matmul(p, dinv, b, w, n, split_out, interpret=False):
  nh = p.shape[1]
  h = w.shape[1]
  bm = 400
  def body(p_ref, d_ref, b_ref, w_ref, *outs):
    o = None
    for hf in range(nh):
      ps = p_ref[0, hf] + p_ref[1, hf]
      hh = jnp.maximum(ps * d_ref[...] + b_ref[:, hf * HALF:(hf + 1) * HALF],
                       0.0)
      t = jnp.dot(hh, w_ref[hf * HALF:(hf + 1) * HALF, :],
                  preferred_element_type=jnp.float32)
      o = t if o is None else o + t
    if split_out:
      outs[0][...] = o[:, 0:HALF]
      outs[1][...] = o[:, HALF:2 * HALF]
    else:
      outs[0][...] = o
  if split_out:
    out_specs = [pl.BlockSpec((bm, HALF), lambda i: (i, 0)),
                 pl.BlockSpec((bm, HALF), lambda i: (i, 0))]
    out_shape = [jax.ShapeDtypeStruct((n, HALF), jnp.float32),
                 jax.ShapeDtypeStruct((n, HALF), jnp.float32)]
  else:
    out_specs = [pl.BlockSpec((bm, h), lambda i: (i, 0))]
    out_shape = [jax.ShapeDtypeStruct((n, h), jnp.float32)]
  return pl.pallas_call(
      body,
      grid=(n // bm,),
      in_specs=[pl.BlockSpec((NC, nh, bm, HALF), lambda i: (0, 0, i, 0)),
                pl.BlockSpec((bm, 1), lambda i: (i, 0)),
                pl.BlockSpec((1, nh * HALF), lambda i: (0, 0)),
                pl.BlockSpec((nh * HALF, h), lambda i: (0, 0))],
      out_specs=out_specs,
      out_shape=out_shape,
      interpret=interpret)(p, dinv, b, w)


def _tc_norm_bias(p, dinv, b, n, interpret=False):
  k = p.shape[3]
  bm = 400
  def body(p_ref, d_ref, b_ref, o_ref):
    o_ref[...] = (p_ref[0, 0] + p_ref[1, 0]) * d_ref[...] + b_ref[...]
  return pl.pallas_call(
      body,
      grid=(n // bm,),
      in_specs=[pl.BlockSpec((NC, 1, bm, k), lambda i: (0, 0, i, 0)),
                pl.BlockSpec((bm, 1), lambda i: (i, 0)),
                pl.BlockSpec((1, k), lambda i: (0, 0))],
      out_specs=pl.BlockSpec((bm, k), lambda i: (i, 0)),
      out_shape=jax.ShapeDtypeStruct((n, k), jnp.float32),
      interpret=interpret)(p, dinv, b)


def _tc_recip(den, n, interpret=False):
  bm = 400
  nw = den.shape[0]
  def body(d_ref, o_ref):
    d = jnp.sum(d_ref[...], axis=0)
    o_ref[...] = jnp.where(d > 0, 1.0 / d, 0.0)
  return pl.pallas_call(
      body,
      grid=(n // bm,),
      in_specs=[pl.BlockSpec((nw, bm, 1), lambda i: (0, i, 0))],
      out_specs=pl.BlockSpec((bm, 1), lambda i: (i, 0)),
      out_shape=jax.ShapeDtypeStruct((n, 1), jnp.float32),
      interpret=interpret)(den)


def _tc_lpa_blend(r, h3, n, interpret=False):
  k = h3.shape[1]
  bm = 400
  def body(r_ref, h_ref, o_ref):
    o_ref[...] = 0.9 * (r_ref[0, 0] + r_ref[1, 0]) + 0.1 * h_ref[...]
  return pl.pallas_call(
      body,
      grid=(n // bm,),
      in_specs=[pl.BlockSpec((NC, 1, bm, k), lambda i: (0, 0, i, 0)),
                pl.BlockSpec((bm, k), lambda i: (i, 0))],
      out_specs=pl.BlockSpec((bm, k), lambda i: (i, 0)),
      out_shape=jax.ShapeDtypeStruct((n, k), jnp.float32),
      interpret=interpret)(r, h3)


def _forward(features, edge_index, lpa_adj, W1, b1, W2, b2, W3, b3,
             interpret=False):
  n, d = features.shape
  e = edge_index.shape[1]
  h = W1.shape[1]
  c = W3.shape[1]

  grain = NW * CB * NIB
  e_pad = ((e + grain - 1) // grain) * grain
  pad = e_pad - e
  src = edge_index[0]
  dst = edge_index[1]
  lv = lpa_adj[:, 0]
  if pad:
    src = jnp.concatenate([src, jnp.zeros((pad,), jnp.int32)])
    dst = jnp.concatenate([dst, jnp.full((pad,), n, jnp.int32)])
    lv = jnp.concatenate([lv, jnp.zeros((pad,), jnp.float32)])
  nch = e_pad // (NW * CB)
  eidx = jnp.stack(
      [src.reshape(nch * NW, CB), dst.reshape(nch * NW, CB),
       lax.bitcast_convert_type(lv, jnp.int32).reshape(nch * NW, CB)],
      axis=1)

  spmv_ex = _make_spmv(n, 2, nch, True, True, interpret)
  spmv_h = _make_spmv(n, 2, nch, True, False, interpret)
  spmv_c = _make_spmv(n, 1, nch, True, False, interpret)
  spmv_raw = _make_spmv(n, 1, nch, False, False, interpret)

  t0a, t0b = _tc_matmul_split(features, W1, interpret)
  p1, den = spmv_ex(t0a, t0b, eidx)
  dinv = _tc_recip(den.reshape(NW, N_PAD, 1)[:, :n], n, interpret)
  t1a, t1b = _tc_norm_relu_matmul(p1, dinv, b1.reshape(1, h), W2, n, True,
                                  interpret)
  p2 = spmv_h(t1a, t1b, eidx)[0]
  t2 = _tc_norm_relu_matmul(p2, dinv, b2.reshape(1, h), W3, n, False,
                            interpret)[0]
  p3 = spmv_c(t2, eidx)[0]
  h3 = _tc_norm_bias(p3, dinv, b3.reshape(1, c), n, interpret)
  r = spmv_raw(h3, eidx)[0]
  z = _tc_lpa_blend(r, h3, n, interpret)
  return h3, z


def kernel(features, edge_index, lpa_adj, W1, b1, W2, b2, W3, b3):
  return _forward(features, edge_index, lpa_adj, W1, b1, W2, b2, W3, b3)


# speedup vs baseline: 1.8752x; 1.5134x over previous
import jax
import jax.numpy as jnp
from jax import lax
from jax.experimental import pallas as pl
from jax.experimental.pallas import tpu as pltpu
from jax.experimental.pallas import tpu_sc as plsc

NC = 2
NS = 16
LANES = 16
NW = NC * NS
CB = 128
NRB = 4
NIB = 8
HALF = 64
N_PAD = 10240
ROWS_PER_SUB = N_PAD // NS
ZCHUNKS = ROWS_PER_SUB // CB


def _sc_mesh():
  return plsc.VectorSubcoreMesh(
      core_axis_name="c", subcore_axis_name="s", num_cores=NC, num_subcores=NS)


def _make_spmv(n_rows, n_halves, chunks_per_worker, exp_coef, emit_den,
               interpret=False):
  nch = chunks_per_worker
  assert nch % NIB == 0
  out_type = [jax.ShapeDtypeStruct((NC, n_halves, N_PAD, HALF), jnp.float32)]
  if emit_den:
    out_type += [jax.ShapeDtypeStruct((NW, N_PAD), jnp.float32)]
  scratch = (
      [pltpu.VMEM((CB, HALF), jnp.float32)] * NRB
      + [pltpu.VMEM((3, CB), jnp.int32)] * NIB
      + [pltpu.VMEM_SHARED((N_PAD, HALF), jnp.float32)]
      + [pltpu.VMEM_SHARED((N_PAD, HALF), jnp.float32)]
      + [pltpu.SemaphoreType.DMA] * (2 * NRB + NIB)
  )
  if emit_den:
    scratch += [pltpu.VMEM((N_PAD,), jnp.float32)]

  def body(*args):
    g_halves = args[0:n_halves]
    eidx_hbm = args[n_halves]
    part_hbm = args[n_halves + 1]
    if emit_den:
      den_hbm = args[n_halves + 2]
      rest = args[n_halves + 3:]
      dpriv = rest[-1]
    else:
      rest = args[n_halves + 2:]
    rows = rest[0:NRB]
    ib = rest[NRB:NRB + NIB]
    gtab = rest[NRB + NIB]
    acc = rest[NRB + NIB + 1]
    s0 = NRB + NIB + 2
    semg = rest[s0:s0 + NRB]
    sems = rest[s0 + NRB:s0 + 2 * NRB]
    semi = rest[s0 + 2 * NRB:s0 + 2 * NRB + NIB]
    rows0 = rows[0]
    cid = lax.axis_index("c")
    sid = lax.axis_index("s")
    wid = sid * NC + cid
    base = wid * nch

    def start_idx(ci, q):
      pltpu.async_copy(eidx_hbm.at[base + ci], ib[q], semi[q])
    def wait_idx(q):
      pltpu.make_async_copy(eidx_hbm.at[base], ib[q], semi[q]).wait()
    def start_gather(b, q):
      pltpu.async_copy(gtab.at[ib[q].at[0]], rows[b], semg[b])
    def wait_gather(b, q):
      pltpu.make_async_copy(gtab.at[ib[q].at[0]], rows[b], semg[b]).wait()
    def start_scatter(b, q):
      pltpu.async_copy(rows[b], acc.at[ib[q].at[1]], sems[b], add=True)
    def wait_scatter(b, q):
      pltpu.make_async_copy(rows[b], acc.at[ib[q].at[1]], sems[b]).wait()

    def scale(b, q, den_flag):
      buf = rows[b]
      idxq = ib[q]
      @plsc.parallel_loop(0, CB // LANES)
      def sgroup(gi):
        sl = pl.ds(gi * LANES, LANES)
        cvec = plsc.bitcast(idxq[2, sl], jnp.float32)
        if exp_coef:
          cvec = jnp.exp(cvec)
        if den_flag:
          plsc.addupdate_scatter(dpriv, [idxq[1, sl]], cvec)
        for i in range(LANES):
          cc = cvec[i]
          for j in range(HALF // LANES):
            fsl = pl.ds(j * LANES, LANES)
            buf[gi * LANES + i, fsl] = buf[gi * LANES + i, fsl] * cc

    def half_pass(half, den_flag):
      g_hbm = g_halves[half]
      def zrow(i, _):
        for j in range(HALF // LANES):
          rows0[i, pl.ds(j * LANES, LANES)] = jnp.zeros((LANES,), jnp.float32)
        return 0
      lax.fori_loop(0, CB, zrow, 0)
      for z in range(ZCHUNKS):
        r0 = sid * ROWS_PER_SUB + z * CB
        pltpu.sync_copy(rows0, acc.at[pl.ds(r0, CB)])
      @pl.when(sid < NS - 1)
      def _():
        r0 = sid * ROWS_PER_SUB
        pltpu.sync_copy(g_hbm.at[pl.ds(r0, ROWS_PER_SUB)],
                        gtab.at[pl.ds(r0, ROWS_PER_SUB)])
      @pl.when(sid == NS - 1)
      def _():
        tail = n_rows - (NS - 1) * ROWS_PER_SUB
        pltpu.sync_copy(g_hbm.at[pl.ds((NS - 1) * ROWS_PER_SUB, tail)],
                        gtab.at[pl.ds((NS - 1) * ROWS_PER_SUB, tail)])
      if den_flag:
        def zd(i, _):
          dpriv[pl.ds(i * LANES, LANES)] = jnp.zeros((LANES,), jnp.float32)
          return 0
        lax.fori_loop(0, N_PAD // LANES, zd, 0)
      plsc.subcore_barrier()

      for q0 in range(5):
        start_idx(q0, q0)
      wait_idx(0)
      start_gather(0, 0)
      ngrp = nch // NIB
      def group(g8, _):
        for r in range(NIB):
          ci = NIB * g8 + r
          b, q = r % NRB, r
          wait_gather(b, q)
          if r == NIB - 1:
            @pl.when(g8 < ngrp - 1)
            def _():
              wait_idx((r + 1) % NIB)
          else:
            wait_idx((r + 1) % NIB)
          if r < 3:
            @pl.when(g8 > 0)
            def _():
              wait_scatter((r - 3) % NRB, (r - 3) % NIB)
          else:
            wait_scatter((r - 3) % NRB, (r - 3) % NIB)
          if r == NIB - 1:
            @pl.when(g8 < ngrp - 1)
            def _():
              start_gather((r + 1) % NRB, (r + 1) % NIB)
          else:
            start_gather((r + 1) % NRB, (r + 1) % NIB)
          if r < 3:
            start_idx(ci + 5, (r + 5) % NIB)
          else:
            @pl.when(g8 < ngrp - 1)
            def _():
              start_idx(ci + 5, (r + 5) % NIB)
          scale(b, q, den_flag)
          start_scatter(b, q)
        return 0
      lax.fori_loop(0, ngrp, group, 0)
      for t in range(3):
        ci = nch - 3 + t
        wait_scatter(ci % NRB, ci % NIB)
      plsc.subcore_barrier()

      for z in range(ZCHUNKS):
        r0 = sid * ROWS_PER_SUB + z * CB
        pltpu.sync_copy(acc.at[pl.ds(r0, CB)], rows0)
        pltpu.sync_copy(rows0, part_hbm.at[cid, half, pl.ds(r0, CB)])

    for half in range(n_halves):
      half_pass(half, emit_den and half == 0)
    if emit_den:
      pltpu.sync_copy(dpriv, den_hbm.at[wid])

  return pl.kernel(body, out_type, mesh=_sc_mesh(), scratch_types=scratch,
                   compiler_params=pltpu.CompilerParams(
                       use_tc_tiling_on_sc=False, needs_layout_passes=False),
                   interpret=interpret)


def _tc_matmul_split(x, w, interpret=False):
  m, d = x.shape
  h = w.shape[1]
  bm = 400
  def body(x_ref, w_ref, oa_ref, ob_ref):
    o = jnp.dot(x_ref[...], w_ref[...], preferred_element_type=jnp.float32)
    oa_ref[...] = o[:, 0:HALF]
    ob_ref[...] = o[:, HALF:2 * HALF]
  return pl.pallas_call(
      body,
      grid=(m // bm,),
      in_specs=[pl.BlockSpec((bm, d), lambda i: (i, 0)),
                pl.BlockSpec((d, h), lambda i: (0, 0))],
      out_specs=[pl.BlockSpec((bm, HALF), lambda i: (i, 0)),
                 pl.BlockSpec((bm, HALF), lambda i: (i, 0))],
      out_shape=[jax.ShapeDtypeStruct((m, HALF), jnp.float32),
                 jax.ShapeDtypeStruct((m, HALF), jnp.float32)],
      interpret=interpret)(x, w)


def _tc_norm_relu_matmul(p, dinv, b, w, n, split_out, interpret=False):
  nh = p.shape[1]
  h = w.shape[1]
  bm = 400
  def body(p_ref, d_ref, b_ref, w_ref, *outs):
    o = None
    for hf in range(nh):
      ps = p_ref[0, hf] + p_ref[1, hf]
      hh = jnp.maximum(ps * d_ref[...] + b_ref[:, hf * HALF:(hf + 1) * HALF],
                       0.0)
      t = jnp.dot(hh, w_ref[hf * HALF:(hf + 1) * HALF, :],
                  preferred_element_type=jnp.float32)
      o = t if o is None else o + t
    if split_out:
      outs[0][...] = o[:, 0:HALF]
      outs[1][...] = o[:, HALF:2 * HALF]
    else:
      outs[0][...] = o
  if split_out:
    out_specs = [pl.BlockSpec((bm, HALF), lambda i: (i, 0)),
                 pl.BlockSpec((bm, HALF), lambda i: (i, 0))]
    out_shape = [jax.ShapeDtypeStruct((n, HALF), jnp.float32),
                 jax.ShapeDtypeStruct((n, HALF), jnp.float32)]
  else:
    out_specs = [pl.BlockSpec((bm, h), lambda i: (i, 0))]
    out_shape = [jax.ShapeDtypeStruct((n, h), jnp.float32)]
  return pl.pallas_call(
      body,
      grid=(n // bm,),
      in_specs=[pl.BlockSpec((NC, nh, bm, HALF), lambda i: (0, 0, i, 0)),
                pl.BlockSpec((bm, 1), lambda i: (i, 0)),
                pl.BlockSpec((1, nh * HALF), lambda i: (0, 0)),
                pl.BlockSpec((nh * HALF, h), lambda i: (0, 0))],
      out_specs=out_specs,
      out_shape=out_shape,
      interpret=interpret)(p, dinv, b, w)


def _tc_norm_bias(p, dinv, b, n, interpret=False):
  k = p.shape[3]
  bm = 400
  def body(p_ref, d_ref, b_ref, o_ref):
    o_ref[...] = (p_ref[0, 0] + p_ref[1, 0]) * d_ref[...] + b_ref[...]
  return pl.pallas_call(
      body,
      grid=(n // bm,),
      in_specs=[pl.BlockSpec((NC, 1, bm, k), lambda i: (0, 0, i, 0)),
                pl.BlockSpec((bm, 1), lambda i: (i, 0)),
                pl.BlockSpec((1, k), lambda i: (0, 0))],
      out_specs=pl.BlockSpec((bm, k), lambda i: (i, 0)),
      out_shape=jax.ShapeDtypeStruct((n, k), jnp.float32),
      interpret=interpret)(p, dinv, b)


def _tc_recip(den, n, interpret=False):
  bm = 400
  nw = den.shape[0]
  def body(d_ref, o_ref):
    d = jnp.sum(d_ref[...], axis=0)
    o_ref[...] = jnp.where(d > 0, 1.0 / d, 0.0)
  return pl.pallas_call(
      body,
      grid=(n // bm,),
      in_specs=[pl.BlockSpec((nw, bm, 1), lambda i: (0, i, 0))],
      out_specs=pl.BlockSpec((bm, 1), lambda i: (i, 0)),
      out_shape=jax.ShapeDtypeStruct((n, 1), jnp.float32),
      interpret=interpret)(den)


def _tc_lpa_blend(r, h3, n, interpret=False):
  k = h3.shape[1]
  bm = 400
  def body(r_ref, h_ref, o_ref):
    o_ref[...] = 0.9 * (r_ref[0, 0] + r_ref[1, 0]) + 0.1 * h_ref[...]
  return pl.pallas_call(
      body,
      grid=(n // bm,),
      in_specs=[pl.BlockSpec((NC, 1, bm, k), lambda i: (0, 0, i, 0)),
                pl.BlockSpec((bm, k), lambda i: (i, 0))],
      out_specs=pl.BlockSpec((bm, k), lambda i: (i, 0)),
      out_shape=jax.ShapeDtypeStruct((n, k), jnp.float32),
      interpret=interpret)(r, h3)


def _forward(features, edge_index, lpa_adj, W1, b1, W2, b2, W3, b3,
             interpret=False):
  n, d = features.shape
  e = edge_index.shape[1]
  h = W1.shape[1]
  c = W3.shape[1]

  grain = NW * CB * NIB
  e_pad = ((e + grain - 1) // grain) * grain
  pad = e_pad - e
  src = edge_index[0]
  dst = edge_index[1]
  lv = lpa_adj[:, 0]
  if pad:
    src = jnp.concatenate([src, jnp.zeros((pad,), jnp.int32)])
    dst = jnp.concatenate([dst, jnp.full((pad,), n, jnp.int32)])
    lv = jnp.concatenate([lv, jnp.zeros((pad,), jnp.float32)])
  nch = e_pad // (NW * CB)
  eidx = jnp.stack(
      [src.reshape(nch * NW, CB), dst.reshape(nch * NW, CB),
       lax.bitcast_convert_type(lv, jnp.int32).reshape(nch * NW, CB)],
      axis=1)

  spmv_ex = _make_spmv(n, 2, nch, True, True, interpret)
  spmv_h = _make_spmv(n, 2, nch, True, False, interpret)
  spmv_c = _make_spmv(n, 1, nch, True, False, interpret)
  spmv_raw = _make_spmv(n, 1, nch, False, False, interpret)

  t0a, t0b = _tc_matmul_split(features, W1, interpret)
  p1, den = spmv_ex(t0a, t0b, eidx)
  dinv = _tc_recip(den.reshape(NW, N_PAD, 1)[:, :n], n, interpret)
  t1a, t1b = _tc_norm_relu_matmul(p1, dinv, b1.reshape(1, h), W2, n, True,
                                  interpret)
  p2 = spmv_h(t1a, t1b, eidx)[0]
  t2 = _tc_norm_relu_matmul(p2, dinv, b2.reshape(1, h), W3, n, False,
                            interpret)[0]
  p3 = spmv_c(t2, eidx)[0]
  h3 = _tc_norm_bias(p3, dinv, b3.reshape(1, c), n, interpret)
  r = spmv_raw(h3, eidx)[0]
  z = _tc_lpa_blend(r, h3, n, interpret)
  return h3, z


def kernel(features, edge_index, lpa_adj, W1, b1, W2, b2, W3, b3):
  return _forward(features, edge_index, lpa_adj, W1, b1, W2, b2, W3, b3)
